# Initial kernel scaffold; baseline (speedup 1.0000x reference)
import jax
import jax.numpy as jnp
from jax.experimental import pallas as pl


def kernel(x, W):
    raise NotImplementedError("write your pallas kernel here")


# speedup vs baseline: 9.5121x; 9.5121x over previous
import functools

import jax
import jax.numpy as jnp
from jax import lax
from jax.experimental import pallas as pl
from jax.experimental.pallas import tpu as pltpu
from jax.experimental.pallas import tpu_sc as plsc

_BETA = 0.25
_N_EMB = 8192
_DIM = 256
_B = 16384

_BM = 512
_BN = 1024


def _argmin_body(x_ref, w_ref, ind_ref, loss_ref, minv_ref, arg_ref):
    n = pl.program_id(1)
    m = pl.program_id(0)
    n_last = pl.num_programs(1) - 1
    m_last = pl.num_programs(0) - 1

    x = x_ref[...]
    w = w_ref[...]
    sx = jnp.sum(x * x, axis=1, keepdims=True)
    sw = jnp.sum(w * w, axis=1)
    dot = lax.dot_general(
        x, w, (((1,), (1,)), ((), ())),
        preferred_element_type=jnp.float32,
    )
    dist = sx - 2.0 * dot + sw[None, :]

    blk_min = jnp.min(dist, axis=1, keepdims=True)
    col = lax.broadcasted_iota(jnp.int32, (_BM, _BN), 1)
    blk_arg = jnp.min(
        jnp.where(dist == blk_min, col, _BN), axis=1, keepdims=True
    ) + n * _BN

    @pl.when(n == 0)
    def _():
        minv_ref[...] = blk_min
        arg_ref[...] = blk_arg

    @pl.when(n > 0)
    def _():
        better = blk_min < minv_ref[...]
        arg_ref[...] = jnp.where(better, blk_arg, arg_ref[...])
        minv_ref[...] = jnp.where(better, blk_min, minv_ref[...])

    @pl.when(n == n_last)
    def _():
        ind_ref[...] = arg_ref[...]
        part = jnp.sum(minv_ref[...])

        @pl.when(m == 0)
        def _():
            loss_ref[0, 0] = part

        @pl.when(m > 0)
        def _():
            loss_ref[0, 0] = loss_ref[0, 0] + part

        @pl.when(m == m_last)
        def _():
            loss_ref[0, 0] = loss_ref[0, 0] * ((1.0 + _BETA) / (_B * _DIM))


_argmin_call = pl.pallas_call(
    _argmin_body,
    grid=(_B // _BM, _N_EMB // _BN),
    in_specs=[
        pl.BlockSpec((_BM, _DIM), lambda m, n: (m, 0)),
        pl.BlockSpec((_BN, _DIM), lambda m, n: (n, 0)),
    ],
    out_specs=[
        pl.BlockSpec((_BM, 1), lambda m, n: (m, 0)),
        pl.BlockSpec((1, 1), lambda m, n: (0, 0), memory_space=pltpu.SMEM),
    ],
    out_shape=[
        jax.ShapeDtypeStruct((_B, 1), jnp.int32),
        jax.ShapeDtypeStruct((1, 1), jnp.float32),
    ],
    scratch_shapes=[
        pltpu.VMEM((_BM, 1), jnp.float32),
        pltpu.VMEM((_BM, 1), jnp.int32),
    ],
)


def _make_gather():
    try:
        info = plsc.get_sparse_core_info()
        nc, ns = info.num_cores, info.num_subcores
    except Exception:
        nc, ns = 2, 16
    nw = nc * ns
    b_per_w = _B // nw
    ch = 128
    n_ch = b_per_w // ch
    mesh = plsc.VectorSubcoreMesh(
        core_axis_name="c", subcore_axis_name="s",
        num_cores=nc, num_subcores=ns,
    )

    @functools.partial(
        pl.kernel,
        out_type=jax.ShapeDtypeStruct((_B, _DIM), jnp.float32),
        mesh=mesh,
        scratch_types=[
            pltpu.VMEM((n_ch, ch), jnp.int32),
            pltpu.VMEM((ch, _DIM), jnp.float32),
            pltpu.VMEM((ch, _DIM), jnp.float32),
            pltpu.SemaphoreType.DMA,
            pltpu.SemaphoreType.DMA,
        ],
    )
    def gather_k(table_hbm, idx_hbm, out_hbm, idx_v, buf0, buf1, sem0, sem1):
        wid = lax.axis_index("s") * nc + lax.axis_index("c")
        base = wid * b_per_w
        pltpu.sync_copy(idx_hbm.at[wid], idx_v)
        bufs = (buf0, buf1)
        sems = (sem0, sem1)
        copies = [None, None]
        copies[0] = pltpu.async_copy(table_hbm.at[idx_v.at[0]], buf0, sem0)
        for j in range(n_ch):
            nxt = (j + 1) % 2
            if j + 1 < n_ch:
                copies[nxt] = pltpu.async_copy(
                    table_hbm.at[idx_v.at[j + 1]], bufs[nxt], sems[nxt]
                )
            copies[j % 2].wait()
            pltpu.sync_copy(
                bufs[j % 2], out_hbm.at[pl.ds(base + j * ch, ch)]
            )

    return gather_k, nw, n_ch, ch


_gather_cache = []


def kernel(x, W):
    if not _gather_cache:
        _gather_cache.append(_make_gather())
    gather_call, nw, n_ch, ch = _gather_cache[0]
    ind2, loss2 = _argmin_call(x, W)
    ind = ind2.reshape(_B)
    quantized = gather_call(W, ind.reshape(nw, n_ch, ch))
    loss = loss2[0, 0]
    return quantized, ind, loss
